# Initial kernel scaffold; baseline (speedup 1.0000x reference)
import dataclasses

import jax
import jax.numpy as jnp
from jax import lax
from jax.experimental import pallas as pl
from jax.experimental.pallas import tpu as pltpu
from jax.experimental.pallas import tpu_sc as plsc

B = 16384
L = 50
D = 32
H = D // 2

NUM_CORES = 2
NUM_SUBCORES = 16
NW = NUM_CORES * NUM_SUBCORES
RPW = B // NW
CHUNK = 64
NCHUNK = RPW // CHUNK
NIDX = CHUNK * L
XFER = 128
NXFER = NIDX // XFER
IDX_PAD = NIDX + 64


def _tree_sum(xs):
    while len(xs) > 1:
        ys = [xs[i] + xs[i + 1] for i in range(0, len(xs) - 1, 2)]
        if len(xs) % 2:
            ys.append(xs[-1])
        xs = ys
    return xs[0]


def _embed_mean_body(words_hbm, table_hbm, out_hbm, idx_v, rows_v, out_v,
                     t0_v, sem):
    wid = lax.axis_index("s") * NUM_CORES + lax.axis_index("c")
    pltpu.sync_copy(table_hbm.at[pl.ds(0, 1)], t0_v)
    t0_lo = t0_v[0, pl.ds(0, H)]
    t0_hi = t0_v[0, pl.ds(H, H)]
    lanes = lax.iota(jnp.int32, 16)
    scale = jnp.float32(1.0 / L)

    @pl.loop(0, NCHUNK)
    def _chunk(c):
        start = wid * (RPW * L) + c * NIDX
        pltpu.sync_copy(words_hbm.at[pl.ds(start, NIDX)],
                        idx_v.at[pl.ds(0, NIDX)])
        copies = [
            pltpu.async_copy(
                table_hbm.at[idx_v.at[pl.ds(j * XFER, XFER)]],
                rows_v.at[pl.ds(j * XFER, XFER)],
                sem,
            )
            for j in range(NXFER)
        ]
        for cp in copies:
            cp.wait()

        @pl.loop(0, CHUNK)
        def _row(i):
            base = i * L
            lo = [rows_v[base + j, pl.ds(0, H)] for j in range(L)]
            hi = [rows_v[base + j, pl.ds(H, H)] for j in range(L)]
            acc_lo = _tree_sum(lo)
            acc_hi = _tree_sum(hi)
            nz = jnp.float32(0.0)
            for q in range(4):
                pos = base + q * 16 + lanes
                if (q + 1) * 16 <= L:
                    vals = plsc.load_gather(idx_v, [pos])
                    hit = vals == 0
                else:
                    live = lanes < jnp.int32(L - q * 16)
                    vals = plsc.load_gather(idx_v, [pos], mask=live)
                    hit = jnp.logical_and(vals == 0, live)
                nz = nz + jnp.sum(jnp.where(hit, jnp.float32(1.0),
                                            jnp.float32(0.0)))
            out_v[i, pl.ds(0, H)] = (acc_lo - nz * t0_lo) * scale
            out_v[i, pl.ds(H, H)] = (acc_hi - nz * t0_hi) * scale

        pltpu.sync_copy(out_v,
                        out_hbm.at[pl.ds(wid * RPW + c * CHUNK, CHUNK)])


def kernel(words, table):
    words_flat = words.reshape(B * L)
    mesh = plsc.VectorSubcoreMesh(core_axis_name="c", subcore_axis_name="s")
    cp = pltpu.CompilerParams()
    if "needs_layout_passes" in pltpu.CompilerParams.__dataclass_fields__:
        cp = dataclasses.replace(cp, needs_layout_passes=False)
    f = pl.kernel(
        _embed_mean_body,
        out_type=jax.ShapeDtypeStruct((B, D), jnp.float32),
        mesh=mesh,
        scratch_types=[
            pltpu.VMEM((IDX_PAD,), jnp.int32),
            pltpu.VMEM((NIDX, D), jnp.float32),
            pltpu.VMEM((CHUNK, D), jnp.float32),
            pltpu.VMEM((1, D), jnp.float32),
            pltpu.SemaphoreType.DMA,
        ],
        compiler_params=cp,
    )
    return f(words_flat, table)


# speedup vs baseline: 2.9131x; 2.9131x over previous
import dataclasses

import jax
import jax.numpy as jnp
from jax import lax
from jax.experimental import pallas as pl
from jax.experimental.pallas import tpu as pltpu
from jax.experimental.pallas import tpu_sc as plsc

B = 16384
L = 50
D = 32
H = D // 2

NUM_CORES = 2
NUM_SUBCORES = 16
NW = NUM_CORES * NUM_SUBCORES
RPW = B // NW
CHUNK = 64
NCHUNK = RPW // CHUNK
NIDX = CHUNK * L
XFER = 128
NXFER = NIDX // XFER
IDX_PAD = NIDX + 64


def _tree_sum(xs):
    while len(xs) > 1:
        ys = [xs[i] + xs[i + 1] for i in range(0, len(xs) - 1, 2)]
        if len(xs) % 2:
            ys.append(xs[-1])
        xs = ys
    return xs[0]


def _embed_mean_body(words_hbm, table_hbm, out_hbm, idx_v, rows_v, out_v,
                     t0_v, sem):
    wid = lax.axis_index("s") * NUM_CORES + lax.axis_index("c")
    pltpu.sync_copy(table_hbm.at[pl.ds(0, 1)], t0_v)
    t0_lo = t0_v[0, pl.ds(0, H)]
    t0_hi = t0_v[0, pl.ds(H, H)]
    lanes = lax.iota(jnp.int32, 16)
    scale = jnp.float32(1.0 / L)

    @pl.loop(0, NCHUNK)
    def _chunk(c):
        start = wid * (RPW * L) + c * NIDX
        pltpu.sync_copy(words_hbm.at[pl.ds(start, NIDX)],
                        idx_v.at[pl.ds(0, NIDX)])
        copies = [
            pltpu.async_copy(
                table_hbm.at[idx_v.at[pl.ds(j * XFER, XFER)]],
                rows_v.at[pl.ds(j * XFER, XFER)],
                sem,
            )
            for j in range(NXFER)
        ]
        for cp in copies:
            cp.wait()

        @pl.loop(0, CHUNK)
        def _row(i):
            base = i * L
            lo = [rows_v[base + j, pl.ds(0, H)] for j in range(L)]
            hi = [rows_v[base + j, pl.ds(H, H)] for j in range(L)]
            acc_lo = _tree_sum(lo)
            acc_hi = _tree_sum(hi)
            nz = jnp.float32(0.0)
            for q in range(4):
                pos = base + q * 16 + lanes
                if (q + 1) * 16 <= L:
                    vals = plsc.load_gather(idx_v, [pos])
                    hit = vals == 0
                else:
                    live = lanes < jnp.int32(L - q * 16)
                    vals = plsc.load_gather(idx_v, [pos], mask=live)
                    hit = jnp.logical_and(vals == 0, live)
                nz = nz + jnp.sum(jnp.where(hit, jnp.float32(1.0),
                                            jnp.float32(0.0)))
            out_v[i, pl.ds(0, H)] = (acc_lo - nz * t0_lo) * scale
            out_v[i, pl.ds(H, H)] = (acc_hi - nz * t0_hi) * scale

        pltpu.sync_copy(out_v,
                        out_hbm.at[pl.ds(wid * RPW + c * CHUNK, CHUNK)])


def kernel(words, table):
    words_flat = words.reshape(B * L)
    mesh = plsc.VectorSubcoreMesh(core_axis_name="c", subcore_axis_name="s")
    cp = pltpu.CompilerParams(use_tc_tiling_on_sc=False)
    if "needs_layout_passes" in pltpu.CompilerParams.__dataclass_fields__:
        cp = dataclasses.replace(cp, needs_layout_passes=False)
    f = pl.kernel(
        _embed_mean_body,
        out_type=jax.ShapeDtypeStruct((B, D), jnp.float32),
        mesh=mesh,
        scratch_types=[
            pltpu.VMEM((IDX_PAD,), jnp.int32),
            pltpu.VMEM((NIDX, D), jnp.float32),
            pltpu.VMEM((CHUNK, D), jnp.float32),
            pltpu.VMEM((1, D), jnp.float32),
            pltpu.SemaphoreType.DMA,
        ],
        compiler_params=cp,
    )
    return f(words_flat, table)
